# baseline (speedup 1.0000x reference)
import functools

import jax
import jax.numpy as jnp
from jax import lax
from jax.experimental import pallas as pl
from jax.experimental.pallas import tpu as pltpu
from jax.experimental.pallas import tpu_sc as plsc

N_NODES = 10000
IN_DIM = 128
H = 4
DH = 32
NEG_SLOPE = 0.2
N_EDGES = 320000

BLK = 128
NBLK = N_EDGES // BLK
NUM_WORKERS = 32
BASE_BLOCKS = NBLK // NUM_WORKERS
EXTRA_TILES = NBLK - BASE_BLOCKS * NUM_WORKERS
CB = 13
NUM_CHUNKS = BASE_BLOCKS // CB

ROW_BLOCK = 1024


def _tc_body(hs_ref, hd_ref, wq_ref, wk_ref, a_ref, elr_ref):
    rows_q = []
    rows_k = []
    for h in range(H):
        al = a_ref[h:h + 1, :DH]
        ar = a_ref[h:h + 1, DH:]
        rows_q.append(jnp.dot(al, wq_ref[h * DH:(h + 1) * DH, :],
                              preferred_element_type=jnp.float32))
        rows_k.append(jnp.dot(ar, wk_ref[h * DH:(h + 1) * DH, :],
                              preferred_element_type=jnp.float32))
    cqt = jnp.concatenate(rows_q, axis=0)
    ckt = jnp.concatenate(rows_k, axis=0)
    elr_ref[:H, :] = lax.dot_general(cqt, hs_ref[...],
                                     (((1,), (1,)), ((), ())),
                                     preferred_element_type=jnp.float32)
    elr_ref[H:, :] = lax.dot_general(ckt, hd_ref[...],
                                     (((1,), (1,)), ((), ())),
                                     preferred_element_type=jnp.float32)


def _node_scores(h_src, h_dst, Wq, Wk, a):
    grid = (N_NODES + ROW_BLOCK - 1) // ROW_BLOCK
    return pl.pallas_call(
        _tc_body,
        grid=(grid,),
        in_specs=[
            pl.BlockSpec((ROW_BLOCK, IN_DIM), lambda i: (i, 0)),
            pl.BlockSpec((ROW_BLOCK, IN_DIM), lambda i: (i, 0)),
            pl.BlockSpec((IN_DIM, IN_DIM), lambda i: (0, 0)),
            pl.BlockSpec((IN_DIM, IN_DIM), lambda i: (0, 0)),
            pl.BlockSpec((H, 2 * DH), lambda i: (0, 0)),
        ],
        out_specs=pl.BlockSpec((2 * H, ROW_BLOCK), lambda i: (0, i)),
        out_shape=jax.ShapeDtypeStruct((2 * H, N_NODES), jnp.float32),
    )(h_src, h_dst, Wq, Wk, a)


_SC_MESH = plsc.VectorSubcoreMesh(core_axis_name="c", subcore_axis_name="s")


@functools.partial(
    pl.kernel,
    out_type=jax.ShapeDtypeStruct((N_EDGES * H,), jnp.float32),
    mesh=_SC_MESH,
    compiler_params=pltpu.CompilerParams(needs_layout_passes=False),
    scratch_types=[
        pltpu.VMEM((2 * H, N_NODES), jnp.float32),
        pltpu.VMEM((2, CB * BLK), jnp.int32),
        pltpu.VMEM((2, CB * BLK), jnp.int32),
        pltpu.VMEM((2, CB * H * BLK), jnp.float32),
        pltpu.VMEM((BLK,), jnp.int32),
        pltpu.VMEM((BLK,), jnp.int32),
        pltpu.VMEM((H * BLK,), jnp.float32),
        pltpu.SemaphoreType.DMA,
        pltpu.SemaphoreType.DMA,
        pltpu.SemaphoreType.DMA,
        pltpu.SemaphoreType.DMA,
        pltpu.SemaphoreType.DMA,
        pltpu.SemaphoreType.DMA,
        pltpu.SemaphoreType.DMA,
    ],
)
def _sc_gather(elr_hbm, ei_hbm, out_hbm, elr_v, src_v, dst_v,
               out_v, tsrc_v, tdst_v, tout_v, sem_tab, sem_i0, sem_i1,
               sem_o0, sem_o1, sem_ti, sem_to):
    cid = lax.axis_index("c")
    sid = lax.axis_index("s")
    wid = sid * 2 + cid
    start_blk = BASE_BLOCKS * wid + jnp.minimum(wid, EXTRA_TILES)

    t_tab = pltpu.async_copy(elr_hbm, elr_v, sem_tab)

    def issue_idx(cb_blk, buf, sem):
        e0 = pl.multiple_of(cb_blk * BLK, 8)
        a = pltpu.async_copy(ei_hbm.at[0, pl.ds(e0, CB * BLK)],
                             src_v.at[buf], sem)
        b = pltpu.async_copy(ei_hbm.at[1, pl.ds(e0, CB * BLK)],
                             dst_v.at[buf], sem)
        return a, b

    i0 = issue_idx(start_blk, 0, sem_i0)
    i1 = issue_idx(start_blk + CB, 1, sem_i1)
    eb = start_blk + BASE_BLOCKS

    @pl.when(wid < EXTRA_TILES)
    def _():
        te0 = pl.multiple_of(eb * BLK, 8)
        pltpu.async_copy(ei_hbm.at[0, pl.ds(te0, BLK)], tsrc_v, sem_ti)
        pltpu.async_copy(ei_hbm.at[1, pl.ds(te0, BLK)], tdst_v, sem_ti)
    t_tab.wait()

    def do_block_g(load_s, load_d, store_y, j, base512):
        for gp in range(BLK // 32):
            vals = []
            for g in (2 * gp, 2 * gp + 1):
                s = load_s(j * BLK + 16 * g)
                d = load_d(j * BLK + 16 * g)
                for h in range(H):
                    hv = jnp.full((16,), h, jnp.int32)
                    hv2 = jnp.full((16,), h + H, jnp.int32)
                    vals.append((g, h,
                                 plsc.load_gather(elr_v, [hv, s]),
                                 plsc.load_gather(elr_v, [hv2, d])))
            for g, h, a, b in vals:
                x = a + b
                y = jnp.maximum(x, NEG_SLOPE * x)
                store_y(base512 + h * BLK + 16 * g, y)

    def do_block(j, buf, base512):
        do_block_g(
            lambda o: src_v[buf, pl.ds(pl.multiple_of(o, 16), 16)],
            lambda o: dst_v[buf, pl.ds(pl.multiple_of(o, 16), 16)],
            lambda o, y: out_v.__setitem__(
                (buf, pl.ds(pl.multiple_of(o, 16), 16)), y),
            j, base512)

    @pl.when(wid < EXTRA_TILES)
    def _():
        pltpu.make_async_copy(ei_hbm.at[0, pl.ds(0, BLK)], tsrc_v,
                              sem_ti).wait()
        pltpu.make_async_copy(ei_hbm.at[1, pl.ds(0, BLK)], tdst_v,
                              sem_ti).wait()
        do_block_g(
            lambda o: tsrc_v[pl.ds(pl.multiple_of(o, 16), 16)],
            lambda o: tdst_v[pl.ds(pl.multiple_of(o, 16), 16)],
            lambda o, y: tout_v.__setitem__(
                pl.ds(pl.multiple_of(o, 16), 16), y),
            0, 0)
        pltpu.async_copy(
            tout_v,
            out_hbm.at[pl.ds(pl.multiple_of(eb * H * BLK, 8), H * BLK)],
            sem_to)

    def compute_chunk(buf):
        @plsc.parallel_loop(0, CB)
        def blk_body(j):
            do_block(j, buf, j * H * BLK)

    def issue_out(cb_blk, buf, sem):
        return pltpu.async_copy(
            out_v.at[buf],
            out_hbm.at[pl.ds(pl.multiple_of(cb_blk * H * BLK, 8), CB * H * BLK)],
            sem)

    def drain_out(buf, sem):
        pltpu.make_async_copy(
            out_v.at[buf],
            out_hbm.at[pl.ds(0, CB * H * BLK)],
            sem).wait()

    def wait_idx(buf, sem):
        pltpu.make_async_copy(ei_hbm.at[0, pl.ds(0, CB * BLK)],
                              src_v.at[buf], sem).wait()
        pltpu.make_async_copy(ei_hbm.at[1, pl.ds(0, CB * BLK)],
                              dst_v.at[buf], sem).wait()

    NPAIRS = NUM_CHUNKS // 2

    def pair_body(k, carry):
        c0 = start_blk + (2 * k) * CB
        c1 = c0 + CB

        wait_idx(0, sem_i0)

        @pl.when(k > 0)
        def _():
            drain_out(0, sem_o0)
        compute_chunk(0)
        issue_out(c0, 0, sem_o0)

        @pl.when(k + 1 < NPAIRS)
        def _():
            issue_idx(c0 + 2 * CB, 0, sem_i0)

        wait_idx(1, sem_i1)

        @pl.when(k > 0)
        def _():
            drain_out(1, sem_o1)
        compute_chunk(1)
        issue_out(c1, 1, sem_o1)

        @pl.when(k + 1 < NPAIRS)
        def _():
            issue_idx(c1 + 2 * CB, 1, sem_i1)
        return carry

    lax.fori_loop(0, NPAIRS, pair_body, 0)
    drain_out(0, sem_o0)
    drain_out(1, sem_o1)

    @pl.when(wid < EXTRA_TILES)
    def _():
        pltpu.make_async_copy(
            tout_v,
            out_hbm.at[pl.ds(0, H * BLK)],
            sem_to).wait()


def kernel(h_src, h_dst, edge_index, Wq, Wk, a):
    elr = _node_scores(h_src, h_dst, Wq, Wk, a)
    out_flat = _sc_gather(elr, edge_index.astype(jnp.int32))
    return (out_flat.reshape(NBLK, H, BLK).transpose(0, 2, 1)
            .reshape(N_EDGES, H))


# speedup vs baseline: 1.0943x; 1.0943x over previous
import functools

import jax
import jax.numpy as jnp
from jax import lax
from jax.experimental import pallas as pl
from jax.experimental.pallas import tpu as pltpu
from jax.experimental.pallas import tpu_sc as plsc

N_NODES = 10000
IN_DIM = 128
H = 4
DH = 32
NEG_SLOPE = 0.2
N_EDGES = 320000

BLK = 128
NBLK = N_EDGES // BLK
NUM_WORKERS = 32
BASE_BLOCKS = NBLK // NUM_WORKERS
EXTRA_TILES = NBLK - BASE_BLOCKS * NUM_WORKERS
CB = 13
NUM_CHUNKS = BASE_BLOCKS // CB

ROW_BLOCK = 2048


def _tc_body(hs_ref, hd_ref, wq_ref, wk_ref, a_ref, elr_ref):
    rows_q = []
    rows_k = []
    for h in range(H):
        al = a_ref[h:h + 1, :DH]
        ar = a_ref[h:h + 1, DH:]
        rows_q.append(jnp.dot(al, wq_ref[h * DH:(h + 1) * DH, :],
                              preferred_element_type=jnp.float32))
        rows_k.append(jnp.dot(ar, wk_ref[h * DH:(h + 1) * DH, :],
                              preferred_element_type=jnp.float32))
    cqt = jnp.concatenate(rows_q, axis=0)
    ckt = jnp.concatenate(rows_k, axis=0)
    elr_ref[:H, :] = lax.dot_general(cqt, hs_ref[...],
                                     (((1,), (1,)), ((), ())),
                                     preferred_element_type=jnp.float32)
    elr_ref[H:, :] = lax.dot_general(ckt, hd_ref[...],
                                     (((1,), (1,)), ((), ())),
                                     preferred_element_type=jnp.float32)


def _node_scores(h_src, h_dst, Wq, Wk, a):
    grid = (N_NODES + ROW_BLOCK - 1) // ROW_BLOCK
    return pl.pallas_call(
        _tc_body,
        grid=(grid,),
        in_specs=[
            pl.BlockSpec((ROW_BLOCK, IN_DIM), lambda i: (i, 0)),
            pl.BlockSpec((ROW_BLOCK, IN_DIM), lambda i: (i, 0)),
            pl.BlockSpec((IN_DIM, IN_DIM), lambda i: (0, 0)),
            pl.BlockSpec((IN_DIM, IN_DIM), lambda i: (0, 0)),
            pl.BlockSpec((H, 2 * DH), lambda i: (0, 0)),
        ],
        out_specs=pl.BlockSpec((2 * H, ROW_BLOCK), lambda i: (0, i)),
        out_shape=jax.ShapeDtypeStruct((2 * H, N_NODES), jnp.float32),
    )(h_src, h_dst, Wq, Wk, a)


_SC_MESH = plsc.VectorSubcoreMesh(core_axis_name="c", subcore_axis_name="s")


@functools.partial(
    pl.kernel,
    out_type=jax.ShapeDtypeStruct((N_EDGES * H,), jnp.float32),
    mesh=_SC_MESH,
    compiler_params=pltpu.CompilerParams(needs_layout_passes=False),
    scratch_types=[
        pltpu.VMEM((2 * H, N_NODES), jnp.float32),
        pltpu.VMEM((2, CB * BLK), jnp.int32),
        pltpu.VMEM((2, CB * BLK), jnp.int32),
        pltpu.VMEM((2, CB * H * BLK), jnp.float32),
        pltpu.VMEM((BLK,), jnp.int32),
        pltpu.VMEM((BLK,), jnp.int32),
        pltpu.VMEM((H * BLK,), jnp.float32),
        pltpu.SemaphoreType.DMA,
        pltpu.SemaphoreType.DMA,
        pltpu.SemaphoreType.DMA,
        pltpu.SemaphoreType.DMA,
        pltpu.SemaphoreType.DMA,
        pltpu.SemaphoreType.DMA,
        pltpu.SemaphoreType.DMA,
    ],
)
def _sc_gather(elr_hbm, ei_hbm, out_hbm, elr_v, src_v, dst_v,
               out_v, tsrc_v, tdst_v, tout_v, sem_tab, sem_i0, sem_i1,
               sem_o0, sem_o1, sem_ti, sem_to):
    cid = lax.axis_index("c")
    sid = lax.axis_index("s")
    wid = sid * 2 + cid
    start_blk = BASE_BLOCKS * wid + jnp.minimum(wid, EXTRA_TILES)

    t_tabs = []
    for i in range(2 * H):
        r = lax.rem(wid + i, 2 * H)
        t_tabs.append(pltpu.async_copy(elr_hbm.at[r], elr_v.at[r], sem_tab))

    def issue_idx(cb_blk, buf, sem):
        e0 = pl.multiple_of(cb_blk * BLK, 8)
        a = pltpu.async_copy(ei_hbm.at[0, pl.ds(e0, CB * BLK)],
                             src_v.at[buf], sem)
        b = pltpu.async_copy(ei_hbm.at[1, pl.ds(e0, CB * BLK)],
                             dst_v.at[buf], sem)
        return a, b

    i0 = issue_idx(start_blk, 0, sem_i0)
    i1 = issue_idx(start_blk + CB, 1, sem_i1)
    eb = start_blk + BASE_BLOCKS

    @pl.when(wid < EXTRA_TILES)
    def _():
        te0 = pl.multiple_of(eb * BLK, 8)
        pltpu.async_copy(ei_hbm.at[0, pl.ds(te0, BLK)], tsrc_v, sem_ti)
        pltpu.async_copy(ei_hbm.at[1, pl.ds(te0, BLK)], tdst_v, sem_ti)
    for t in t_tabs:
        t.wait()

    def do_block_g(load_s, load_d, store_y, j, base512):
        for gp in range(BLK // 32):
            vals = []
            for g in (2 * gp, 2 * gp + 1):
                s = load_s(j * BLK + 16 * g)
                d = load_d(j * BLK + 16 * g)
                for h in range(H):
                    hv = jnp.full((16,), h, jnp.int32)
                    hv2 = jnp.full((16,), h + H, jnp.int32)
                    vals.append((g, h,
                                 plsc.load_gather(elr_v, [hv, s]),
                                 plsc.load_gather(elr_v, [hv2, d])))
            for g, h, a, b in vals:
                x = a + b
                y = jnp.maximum(x, NEG_SLOPE * x)
                store_y(base512 + h * BLK + 16 * g, y)

    def do_block(j, buf, base512):
        do_block_g(
            lambda o: src_v[buf, pl.ds(pl.multiple_of(o, 16), 16)],
            lambda o: dst_v[buf, pl.ds(pl.multiple_of(o, 16), 16)],
            lambda o, y: out_v.__setitem__(
                (buf, pl.ds(pl.multiple_of(o, 16), 16)), y),
            j, base512)

    @pl.when(wid < EXTRA_TILES)
    def _():
        pltpu.make_async_copy(ei_hbm.at[0, pl.ds(0, BLK)], tsrc_v,
                              sem_ti).wait()
        pltpu.make_async_copy(ei_hbm.at[1, pl.ds(0, BLK)], tdst_v,
                              sem_ti).wait()
        do_block_g(
            lambda o: tsrc_v[pl.ds(pl.multiple_of(o, 16), 16)],
            lambda o: tdst_v[pl.ds(pl.multiple_of(o, 16), 16)],
            lambda o, y: tout_v.__setitem__(
                pl.ds(pl.multiple_of(o, 16), 16), y),
            0, 0)
        pltpu.async_copy(
            tout_v,
            out_hbm.at[pl.ds(pl.multiple_of(eb * H * BLK, 8), H * BLK)],
            sem_to)

    def compute_chunk(buf):
        @plsc.parallel_loop(0, CB)
        def blk_body(j):
            do_block(j, buf, j * H * BLK)

    def issue_out(cb_blk, buf, sem):
        return pltpu.async_copy(
            out_v.at[buf],
            out_hbm.at[pl.ds(pl.multiple_of(cb_blk * H * BLK, 8), CB * H * BLK)],
            sem)

    def drain_out(buf, sem):
        pltpu.make_async_copy(
            out_v.at[buf],
            out_hbm.at[pl.ds(0, CB * H * BLK)],
            sem).wait()

    def wait_idx(buf, sem):
        pltpu.make_async_copy(ei_hbm.at[0, pl.ds(0, CB * BLK)],
                              src_v.at[buf], sem).wait()
        pltpu.make_async_copy(ei_hbm.at[1, pl.ds(0, CB * BLK)],
                              dst_v.at[buf], sem).wait()

    NPAIRS = NUM_CHUNKS // 2

    def pair_body(k, carry):
        c0 = start_blk + (2 * k) * CB
        c1 = c0 + CB

        wait_idx(0, sem_i0)

        @pl.when(k > 0)
        def _():
            drain_out(0, sem_o0)
        compute_chunk(0)
        issue_out(c0, 0, sem_o0)

        @pl.when(k + 1 < NPAIRS)
        def _():
            issue_idx(c0 + 2 * CB, 0, sem_i0)

        wait_idx(1, sem_i1)

        @pl.when(k > 0)
        def _():
            drain_out(1, sem_o1)
        compute_chunk(1)
        issue_out(c1, 1, sem_o1)

        @pl.when(k + 1 < NPAIRS)
        def _():
            issue_idx(c1 + 2 * CB, 1, sem_i1)
        return carry

    lax.fori_loop(0, NPAIRS, pair_body, 0)
    drain_out(0, sem_o0)
    drain_out(1, sem_o1)

    @pl.when(wid < EXTRA_TILES)
    def _():
        pltpu.make_async_copy(
            tout_v,
            out_hbm.at[pl.ds(0, H * BLK)],
            sem_to).wait()


def kernel(h_src, h_dst, edge_index, Wq, Wk, a):
    elr = _node_scores(h_src, h_dst, Wq, Wk, a)
    out_flat = _sc_gather(elr, edge_index.astype(jnp.int32))
    return (out_flat.reshape(NBLK, H, BLK).transpose(0, 2, 1)
            .reshape(N_EDGES, H))


# speedup vs baseline: 1.1080x; 1.0125x over previous
import functools

import jax
import jax.numpy as jnp
from jax import lax
from jax.experimental import pallas as pl
from jax.experimental.pallas import tpu as pltpu
from jax.experimental.pallas import tpu_sc as plsc

N_NODES = 10000
IN_DIM = 128
H = 4
DH = 32
NEG_SLOPE = 0.2
N_EDGES = 320000

BLK = 128
NBLK = N_EDGES // BLK
NUM_WORKERS = 32
BASE_BLOCKS = NBLK // NUM_WORKERS
EXTRA_TILES = NBLK - BASE_BLOCKS * NUM_WORKERS
CB = 13
NUM_CHUNKS = BASE_BLOCKS // CB

ROW_BLOCK = 2048


def _tc_body(hs_ref, hd_ref, wq_ref, wk_ref, a_ref, elr_ref):
    rows_q = []
    rows_k = []
    for h in range(H):
        al = a_ref[h:h + 1, :DH]
        ar = a_ref[h:h + 1, DH:]
        rows_q.append(jnp.dot(al, wq_ref[h * DH:(h + 1) * DH, :],
                              preferred_element_type=jnp.float32))
        rows_k.append(jnp.dot(ar, wk_ref[h * DH:(h + 1) * DH, :],
                              preferred_element_type=jnp.float32))
    cqt = jnp.concatenate(rows_q, axis=0)
    ckt = jnp.concatenate(rows_k, axis=0)
    elr_ref[:H, :] = lax.dot_general(cqt, hs_ref[...],
                                     (((1,), (1,)), ((), ())),
                                     preferred_element_type=jnp.float32)
    elr_ref[H:, :] = lax.dot_general(ckt, hd_ref[...],
                                     (((1,), (1,)), ((), ())),
                                     preferred_element_type=jnp.float32)


def _node_scores(h_src, h_dst, Wq, Wk, a):
    grid = (N_NODES + ROW_BLOCK - 1) // ROW_BLOCK
    return pl.pallas_call(
        _tc_body,
        grid=(grid,),
        in_specs=[
            pl.BlockSpec((ROW_BLOCK, IN_DIM), lambda i: (i, 0)),
            pl.BlockSpec((ROW_BLOCK, IN_DIM), lambda i: (i, 0)),
            pl.BlockSpec((IN_DIM, IN_DIM), lambda i: (0, 0)),
            pl.BlockSpec((IN_DIM, IN_DIM), lambda i: (0, 0)),
            pl.BlockSpec((H, 2 * DH), lambda i: (0, 0)),
        ],
        out_specs=pl.BlockSpec((2 * H, ROW_BLOCK), lambda i: (0, i)),
        out_shape=jax.ShapeDtypeStruct((2 * H, N_NODES), jnp.float32),
    )(h_src, h_dst, Wq, Wk, a)


_SC_MESH = plsc.VectorSubcoreMesh(core_axis_name="c", subcore_axis_name="s")


@functools.partial(
    pl.kernel,
    out_type=jax.ShapeDtypeStruct((N_EDGES * H,), jnp.float32),
    mesh=_SC_MESH,
    compiler_params=pltpu.CompilerParams(needs_layout_passes=False),
    scratch_types=[
        pltpu.VMEM((2 * H, N_NODES), jnp.float32),
        pltpu.VMEM((2, CB * BLK), jnp.int32),
        pltpu.VMEM((2, CB * BLK), jnp.int32),
        pltpu.VMEM((2, CB * H * BLK), jnp.float32),
        pltpu.VMEM((BLK,), jnp.int32),
        pltpu.VMEM((BLK,), jnp.int32),
        pltpu.VMEM((H * BLK,), jnp.float32),
        pltpu.SemaphoreType.DMA,
        pltpu.SemaphoreType.DMA,
        pltpu.SemaphoreType.DMA,
        pltpu.SemaphoreType.DMA,
        pltpu.SemaphoreType.DMA,
        pltpu.SemaphoreType.DMA,
        pltpu.SemaphoreType.DMA,
    ],
)
def _sc_gather(elr_hbm, ei_hbm, out_hbm, elr_v, src_v, dst_v,
               out_v, tsrc_v, tdst_v, tout_v, sem_tab, sem_i0, sem_i1,
               sem_o0, sem_o1, sem_ti, sem_to):
    cid = lax.axis_index("c")
    sid = lax.axis_index("s")
    wid = sid * 2 + cid
    start_blk = BASE_BLOCKS * wid + jnp.minimum(wid, EXTRA_TILES)

    t_tabs = []
    for i in range(2 * H):
        r = lax.rem(wid + i, 2 * H)
        t_tabs.append(pltpu.async_copy(elr_hbm.at[r], elr_v.at[r], sem_tab))

    def issue_idx(cb_blk, buf, sem):
        e0 = pl.multiple_of(cb_blk * BLK, 8)
        a = pltpu.async_copy(ei_hbm.at[0, pl.ds(e0, CB * BLK)],
                             src_v.at[buf], sem)
        b = pltpu.async_copy(ei_hbm.at[1, pl.ds(e0, CB * BLK)],
                             dst_v.at[buf], sem)
        return a, b

    i0 = issue_idx(start_blk, 0, sem_i0)
    i1 = issue_idx(start_blk + CB, 1, sem_i1)
    eb = start_blk + BASE_BLOCKS

    @pl.when(wid < EXTRA_TILES)
    def _():
        te0 = pl.multiple_of(eb * BLK, 8)
        pltpu.async_copy(ei_hbm.at[0, pl.ds(te0, BLK)], tsrc_v, sem_ti)
        pltpu.async_copy(ei_hbm.at[1, pl.ds(te0, BLK)], tdst_v, sem_ti)
    for t in t_tabs:
        t.wait()

    def do_block_g(load_s, load_d, store_y, j, base512):
        for gp in range(BLK // 64):
            vals = []
            for g in (4 * gp, 4 * gp + 1, 4 * gp + 2, 4 * gp + 3):
                s = load_s(j * BLK + 16 * g)
                d = load_d(j * BLK + 16 * g)
                for h in range(H):
                    hv = jnp.full((16,), h, jnp.int32)
                    hv2 = jnp.full((16,), h + H, jnp.int32)
                    vals.append((g, h,
                                 plsc.load_gather(elr_v, [hv, s]),
                                 plsc.load_gather(elr_v, [hv2, d])))
            for g, h, a, b in vals:
                x = a + b
                y = jnp.maximum(x, NEG_SLOPE * x)
                store_y(base512 + h * BLK + 16 * g, y)

    def do_block(j, buf, base512):
        do_block_g(
            lambda o: src_v[buf, pl.ds(pl.multiple_of(o, 16), 16)],
            lambda o: dst_v[buf, pl.ds(pl.multiple_of(o, 16), 16)],
            lambda o, y: out_v.__setitem__(
                (buf, pl.ds(pl.multiple_of(o, 16), 16)), y),
            j, base512)

    @pl.when(wid < EXTRA_TILES)
    def _():
        pltpu.make_async_copy(ei_hbm.at[0, pl.ds(0, BLK)], tsrc_v,
                              sem_ti).wait()
        pltpu.make_async_copy(ei_hbm.at[1, pl.ds(0, BLK)], tdst_v,
                              sem_ti).wait()
        do_block_g(
            lambda o: tsrc_v[pl.ds(pl.multiple_of(o, 16), 16)],
            lambda o: tdst_v[pl.ds(pl.multiple_of(o, 16), 16)],
            lambda o, y: tout_v.__setitem__(
                pl.ds(pl.multiple_of(o, 16), 16), y),
            0, 0)
        pltpu.async_copy(
            tout_v,
            out_hbm.at[pl.ds(pl.multiple_of(eb * H * BLK, 8), H * BLK)],
            sem_to)

    def compute_chunk(buf):
        @plsc.parallel_loop(0, CB)
        def blk_body(j):
            do_block(j, buf, j * H * BLK)

    def issue_out(cb_blk, buf, sem):
        return pltpu.async_copy(
            out_v.at[buf],
            out_hbm.at[pl.ds(pl.multiple_of(cb_blk * H * BLK, 8), CB * H * BLK)],
            sem)

    def drain_out(buf, sem):
        pltpu.make_async_copy(
            out_v.at[buf],
            out_hbm.at[pl.ds(0, CB * H * BLK)],
            sem).wait()

    def wait_idx(buf, sem):
        pltpu.make_async_copy(ei_hbm.at[0, pl.ds(0, CB * BLK)],
                              src_v.at[buf], sem).wait()
        pltpu.make_async_copy(ei_hbm.at[1, pl.ds(0, CB * BLK)],
                              dst_v.at[buf], sem).wait()

    NPAIRS = NUM_CHUNKS // 2

    def pair_body(k, carry):
        c0 = start_blk + (2 * k) * CB
        c1 = c0 + CB

        wait_idx(0, sem_i0)

        @pl.when(k > 0)
        def _():
            drain_out(0, sem_o0)
        compute_chunk(0)
        issue_out(c0, 0, sem_o0)

        @pl.when(k + 1 < NPAIRS)
        def _():
            issue_idx(c0 + 2 * CB, 0, sem_i0)

        wait_idx(1, sem_i1)

        @pl.when(k > 0)
        def _():
            drain_out(1, sem_o1)
        compute_chunk(1)
        issue_out(c1, 1, sem_o1)

        @pl.when(k + 1 < NPAIRS)
        def _():
            issue_idx(c1 + 2 * CB, 1, sem_i1)
        return carry

    lax.fori_loop(0, NPAIRS, pair_body, 0)
    drain_out(0, sem_o0)
    drain_out(1, sem_o1)

    @pl.when(wid < EXTRA_TILES)
    def _():
        pltpu.make_async_copy(
            tout_v,
            out_hbm.at[pl.ds(0, H * BLK)],
            sem_to).wait()


def kernel(h_src, h_dst, edge_index, Wq, Wk, a):
    elr = _node_scores(h_src, h_dst, Wq, Wk, a)
    out_flat = _sc_gather(elr, edge_index.astype(jnp.int32))
    return (out_flat.reshape(NBLK, H, BLK).transpose(0, 2, 1)
            .reshape(N_EDGES, H))


# speedup vs baseline: 1.1181x; 1.0091x over previous
import functools

import jax
import jax.numpy as jnp
from jax import lax
from jax.experimental import pallas as pl
from jax.experimental.pallas import tpu as pltpu
from jax.experimental.pallas import tpu_sc as plsc

N_NODES = 10000
IN_DIM = 128
H = 4
DH = 32
NEG_SLOPE = 0.2
N_EDGES = 320000

BLK = 128
NBLK = N_EDGES // BLK
NUM_WORKERS = 32
BASE_BLOCKS = NBLK // NUM_WORKERS
EXTRA_TILES = NBLK - BASE_BLOCKS * NUM_WORKERS
CB = 13
NUM_CHUNKS = BASE_BLOCKS // CB

ROW_BLOCK = 2048


def _tc_body(hs_ref, hd_ref, wq_ref, wk_ref, a_ref, elr_ref):
    rows_q = []
    rows_k = []
    for h in range(H):
        al = a_ref[h:h + 1, :DH]
        ar = a_ref[h:h + 1, DH:]
        rows_q.append(jnp.dot(al, wq_ref[h * DH:(h + 1) * DH, :],
                              preferred_element_type=jnp.float32))
        rows_k.append(jnp.dot(ar, wk_ref[h * DH:(h + 1) * DH, :],
                              preferred_element_type=jnp.float32))
    cqt = jnp.concatenate(rows_q, axis=0)
    ckt = jnp.concatenate(rows_k, axis=0)
    elr_ref[:H, :] = lax.dot_general(cqt, hs_ref[...],
                                     (((1,), (1,)), ((), ())),
                                     preferred_element_type=jnp.float32)
    elr_ref[H:, :] = lax.dot_general(ckt, hd_ref[...],
                                     (((1,), (1,)), ((), ())),
                                     preferred_element_type=jnp.float32)


def _node_scores(h_src, h_dst, Wq, Wk, a):
    grid = (N_NODES + ROW_BLOCK - 1) // ROW_BLOCK
    return pl.pallas_call(
        _tc_body,
        grid=(grid,),
        in_specs=[
            pl.BlockSpec((ROW_BLOCK, IN_DIM), lambda i: (i, 0)),
            pl.BlockSpec((ROW_BLOCK, IN_DIM), lambda i: (i, 0)),
            pl.BlockSpec((IN_DIM, IN_DIM), lambda i: (0, 0)),
            pl.BlockSpec((IN_DIM, IN_DIM), lambda i: (0, 0)),
            pl.BlockSpec((H, 2 * DH), lambda i: (0, 0)),
        ],
        out_specs=pl.BlockSpec((2 * H, ROW_BLOCK), lambda i: (0, i)),
        out_shape=jax.ShapeDtypeStruct((2 * H, N_NODES), jnp.float32),
    )(h_src, h_dst, Wq, Wk, a)


_SC_MESH = plsc.VectorSubcoreMesh(core_axis_name="c", subcore_axis_name="s")


@functools.partial(
    pl.kernel,
    out_type=jax.ShapeDtypeStruct((N_EDGES * H,), jnp.float32),
    mesh=_SC_MESH,
    compiler_params=pltpu.CompilerParams(needs_layout_passes=False),
    scratch_types=[
        pltpu.VMEM((2 * H, N_NODES), jnp.float32),
        pltpu.VMEM((2, CB * BLK), jnp.int32),
        pltpu.VMEM((2, CB * BLK), jnp.int32),
        pltpu.VMEM((2, CB * H * BLK), jnp.float32),
        pltpu.VMEM((BLK,), jnp.int32),
        pltpu.VMEM((BLK,), jnp.int32),
        pltpu.VMEM((H * BLK,), jnp.float32),
        pltpu.SemaphoreType.DMA,
        pltpu.SemaphoreType.DMA,
        pltpu.SemaphoreType.DMA,
        pltpu.SemaphoreType.DMA,
        pltpu.SemaphoreType.DMA,
        pltpu.SemaphoreType.DMA,
        pltpu.SemaphoreType.DMA,
    ],
)
def _sc_gather(elr_hbm, ei_hbm, out_hbm, elr_v, src_v, dst_v,
               out_v, tsrc_v, tdst_v, tout_v, sem_tab, sem_i0, sem_i1,
               sem_o0, sem_o1, sem_ti, sem_to):
    cid = lax.axis_index("c")
    sid = lax.axis_index("s")
    wid = sid * 2 + cid
    start_blk = BASE_BLOCKS * wid + jnp.minimum(wid, EXTRA_TILES)

    t_tabs = []
    for i in range(2 * H):
        r = lax.rem(wid + i, 2 * H)
        t_tabs.append(pltpu.async_copy(elr_hbm.at[r], elr_v.at[r], sem_tab))

    def issue_idx(cb_blk, buf, sem):
        e0 = pl.multiple_of(cb_blk * BLK, 8)
        a = pltpu.async_copy(ei_hbm.at[0, pl.ds(e0, CB * BLK)],
                             src_v.at[buf], sem)
        b = pltpu.async_copy(ei_hbm.at[1, pl.ds(e0, CB * BLK)],
                             dst_v.at[buf], sem)
        return a, b

    i0 = issue_idx(start_blk, 0, sem_i0)
    i1 = issue_idx(start_blk + CB, 1, sem_i1)
    eb = start_blk + BASE_BLOCKS

    @pl.when(wid < EXTRA_TILES)
    def _():
        te0 = pl.multiple_of(eb * BLK, 8)
        pltpu.async_copy(ei_hbm.at[0, pl.ds(te0, BLK)], tsrc_v, sem_ti)
        pltpu.async_copy(ei_hbm.at[1, pl.ds(te0, BLK)], tdst_v, sem_ti)
    for t in t_tabs:
        t.wait()

    def do_block_g(load_s, load_d, store_y, j, base512):
        for gp in range(BLK // 128):
            vals = []
            for g in range(8 * gp, 8 * gp + 8):
                s = load_s(j * BLK + 16 * g)
                d = load_d(j * BLK + 16 * g)
                for h in range(H):
                    hv = jnp.full((16,), h, jnp.int32)
                    hv2 = jnp.full((16,), h + H, jnp.int32)
                    vals.append((g, h,
                                 plsc.load_gather(elr_v, [hv, s]),
                                 plsc.load_gather(elr_v, [hv2, d])))
            for g, h, a, b in vals:
                x = a + b
                y = jnp.maximum(x, NEG_SLOPE * x)
                store_y(base512 + h * BLK + 16 * g, y)

    def do_block(j, buf, base512):
        do_block_g(
            lambda o: src_v[buf, pl.ds(pl.multiple_of(o, 16), 16)],
            lambda o: dst_v[buf, pl.ds(pl.multiple_of(o, 16), 16)],
            lambda o, y: out_v.__setitem__(
                (buf, pl.ds(pl.multiple_of(o, 16), 16)), y),
            j, base512)

    @pl.when(wid < EXTRA_TILES)
    def _():
        pltpu.make_async_copy(ei_hbm.at[0, pl.ds(0, BLK)], tsrc_v,
                              sem_ti).wait()
        pltpu.make_async_copy(ei_hbm.at[1, pl.ds(0, BLK)], tdst_v,
                              sem_ti).wait()
        do_block_g(
            lambda o: tsrc_v[pl.ds(pl.multiple_of(o, 16), 16)],
            lambda o: tdst_v[pl.ds(pl.multiple_of(o, 16), 16)],
            lambda o, y: tout_v.__setitem__(
                pl.ds(pl.multiple_of(o, 16), 16), y),
            0, 0)
        pltpu.async_copy(
            tout_v,
            out_hbm.at[pl.ds(pl.multiple_of(eb * H * BLK, 8), H * BLK)],
            sem_to)

    def compute_chunk(buf):
        @plsc.parallel_loop(0, CB)
        def blk_body(j):
            do_block(j, buf, j * H * BLK)

    def issue_out(cb_blk, buf, sem):
        return pltpu.async_copy(
            out_v.at[buf],
            out_hbm.at[pl.ds(pl.multiple_of(cb_blk * H * BLK, 8), CB * H * BLK)],
            sem)

    def drain_out(buf, sem):
        pltpu.make_async_copy(
            out_v.at[buf],
            out_hbm.at[pl.ds(0, CB * H * BLK)],
            sem).wait()

    def wait_idx(buf, sem):
        pltpu.make_async_copy(ei_hbm.at[0, pl.ds(0, CB * BLK)],
                              src_v.at[buf], sem).wait()
        pltpu.make_async_copy(ei_hbm.at[1, pl.ds(0, CB * BLK)],
                              dst_v.at[buf], sem).wait()

    NPAIRS = NUM_CHUNKS // 2

    def pair_body(k, carry):
        c0 = start_blk + (2 * k) * CB
        c1 = c0 + CB

        wait_idx(0, sem_i0)

        @pl.when(k > 0)
        def _():
            drain_out(0, sem_o0)
        compute_chunk(0)
        issue_out(c0, 0, sem_o0)

        @pl.when(k + 1 < NPAIRS)
        def _():
            issue_idx(c0 + 2 * CB, 0, sem_i0)

        wait_idx(1, sem_i1)

        @pl.when(k > 0)
        def _():
            drain_out(1, sem_o1)
        compute_chunk(1)
        issue_out(c1, 1, sem_o1)

        @pl.when(k + 1 < NPAIRS)
        def _():
            issue_idx(c1 + 2 * CB, 1, sem_i1)
        return carry

    lax.fori_loop(0, NPAIRS, pair_body, 0)
    drain_out(0, sem_o0)
    drain_out(1, sem_o1)

    @pl.when(wid < EXTRA_TILES)
    def _():
        pltpu.make_async_copy(
            tout_v,
            out_hbm.at[pl.ds(0, H * BLK)],
            sem_to).wait()


def kernel(h_src, h_dst, edge_index, Wq, Wk, a):
    elr = _node_scores(h_src, h_dst, Wq, Wk, a)
    out_flat = _sc_gather(elr, edge_index.astype(jnp.int32))
    return (out_flat.reshape(NBLK, H, BLK).transpose(0, 2, 1)
            .reshape(N_EDGES, H))
